# baseline (speedup 1.0000x reference)
import functools

import jax
import jax.numpy as jnp
from jax import lax
from jax.experimental import pallas as pl
from jax.experimental.pallas import tpu as pltpu
from jax.experimental.pallas import tpu_sc as plsc

BATCH = 4096
SEQ_LEN = 200
EMB = 64
TOTAL = BATCH * SEQ_LEN

NC = 2
NS = 16
NW = NC * NS
ROWS_PER_W = TOTAL // NW
CHUNK = 128
NCH = ROWS_PER_W // CHUNK
NBUF = 4
SB = BATCH // CHUNK
LANES = 16


def _make_embed():
    mesh = plsc.VectorSubcoreMesh(core_axis_name="c", subcore_axis_name="s")

    @functools.partial(
        pl.kernel,
        mesh=mesh,
        out_type=jax.ShapeDtypeStruct((SEQ_LEN, EMB, BATCH), jnp.float32),
        scratch_types=[
            pltpu.VMEM((ROWS_PER_W,), jnp.int32),
            pltpu.VMEM((NBUF, CHUNK, EMB), jnp.float32),
            pltpu.VMEM((NBUF, EMB, CHUNK), jnp.float32),
            pltpu.SemaphoreType.DMA((NBUF,)),
            pltpu.SemaphoreType.DMA((NBUF,)),
        ],
        compiler_params=pltpu.CompilerParams(
            use_tc_tiling_on_sc=False, needs_layout_passes=False
        ),
    )
    def embed(table_hbm, idx_hbm, out_hbm, idx_v, wbufs, tbufs, gsem, wsem):
        wid = lax.axis_index("s") * NC + lax.axis_index("c")
        pltpu.sync_copy(idx_hbm.at[pl.ds(wid * ROWS_PER_W, ROWS_PER_W)], idx_v)
        q0 = wid * NCH

        def gather(c, b):
            return pltpu.make_async_copy(
                table_hbm.at[idx_v.at[pl.ds(c * CHUNK, CHUNK)]],
                wbufs.at[b],
                gsem.at[b],
            )

        def write(c, b):
            q = q0 + c
            s = 8 * (q // (SB * 8)) + (q % 8)
            b0 = CHUNK * ((q % (SB * 8)) // 8)
            return pltpu.make_async_copy(
                tbufs.at[b],
                out_hbm.at[s, :, pl.ds(b0, CHUNK)],
                wsem.at[b],
            )

        row_ids = [lax.iota(jnp.int32, LANES) + g * LANES
                   for g in range(CHUNK // LANES)]
        zeros16 = jnp.zeros((LANES,), jnp.int32)

        def transpose_chunk(b):
            def erow(e, carry):
                cols = zeros16 + e
                for g in range(CHUNK // LANES):
                    vals = plsc.load_gather(wbufs.at[b], [row_ids[g], cols])
                    tbufs[b, e, pl.ds(g * LANES, LANES)] = vals
                return carry

            lax.fori_loop(0, EMB, erow, 0)

        for b in range(NBUF):
            gather(b, b).start()

        def round_body(r, carry):
            for b in range(NBUF):
                c = r * NBUF + b
                gather(c, b).wait()

                @pl.when(r > 0)
                def _():
                    write(c - NBUF, b).wait()

                transpose_chunk(b)
                write(c, b).start()

                @pl.when(c + NBUF < NCH)
                def _():
                    gather(c + NBUF, b).start()

            return carry

        lax.fori_loop(0, NCH // NBUF, round_body, 0)
        for b in range(NBUF):
            write(NCH - NBUF + b, b).wait()

    return embed


_embed = _make_embed()


def kernel(inputs, embedding_table):
    idx = (inputs.T.reshape(SEQ_LEN // 8, 8, SB, CHUNK)
           .transpose(0, 2, 1, 3).reshape(TOTAL).astype(jnp.int32))
    out_seb = _embed(embedding_table, idx)
    return jnp.transpose(out_seb, (2, 0, 1))


# speedup vs baseline: 1.1274x; 1.1274x over previous
import functools

import jax
import jax.numpy as jnp
from jax import lax
from jax.experimental import pallas as pl
from jax.experimental.pallas import tpu as pltpu
from jax.experimental.pallas import tpu_sc as plsc

BATCH = 4096
SEQ_LEN = 200
EMB = 64
TOTAL = BATCH * SEQ_LEN
VOCAB_ROWS = 1000000

NC = 2
NS = 16
NW = NC * NS
ROWS_PER_W = TOTAL // NW
CHUNK = 128
NCH = ROWS_PER_W // CHUNK
NBUF = 4
SB = BATCH // CHUNK
LANES = 16


def _make_embed():
    mesh = plsc.VectorSubcoreMesh(core_axis_name="c", subcore_axis_name="s")

    @functools.partial(
        pl.kernel,
        mesh=mesh,
        out_type=jax.ShapeDtypeStruct((SEQ_LEN, EMB, BATCH), jnp.float32),
        scratch_types=[
            pltpu.VMEM((ROWS_PER_W,), jnp.int32),
            pltpu.VMEM((NBUF, CHUNK, EMB), jnp.float32),
            pltpu.VMEM((NBUF, EMB, CHUNK), jnp.float32),
            pltpu.SemaphoreType.DMA((NBUF,)),
            pltpu.SemaphoreType.DMA((NBUF,)),
        ],
        compiler_params=pltpu.CompilerParams(
            use_tc_tiling_on_sc=False, needs_layout_passes=False
        ),
    )
    def embed(table_hbm, idx_hbm, out_hbm, idx_v, wbufs, tbufs, gsem, wsem):
        wid = lax.axis_index("s") * NC + lax.axis_index("c")
        pltpu.sync_copy(idx_hbm.at[pl.ds(wid * ROWS_PER_W, ROWS_PER_W)], idx_v)
        q0 = wid * NCH

        def gather(c, b):
            return pltpu.make_async_copy(
                table_hbm.at[idx_v.at[pl.ds(c * CHUNK, CHUNK)]],
                wbufs.at[b],
                gsem.at[b],
            )

        def write(c, b):
            q = q0 + c
            s = 8 * (q // (SB * 8)) + (q % 8)
            b0 = CHUNK * ((q % (SB * 8)) // 8)
            return pltpu.make_async_copy(
                tbufs.at[b],
                out_hbm.at[s, :, pl.ds(b0, CHUNK)],
                wsem.at[b],
            )

        iota16 = lax.iota(jnp.int32, LANES)
        e_rows = [iota16 + q * LANES for q in range(EMB // LANES)]
        zeros16 = jnp.zeros((LANES,), jnp.int32)

        def transpose_chunk(b):
            def jrow(j, carry):
                cols = zeros16 + j
                for q in range(EMB // LANES):
                    vals = wbufs[b, j, pl.ds(q * LANES, LANES)]
                    plsc.store_scatter(tbufs.at[b], [e_rows[q], cols], vals)
                return carry

            lax.fori_loop(0, CHUNK, jrow, 0)

        for b in range(NBUF):
            gather(b, b).start()

        def round_body(r, carry):
            for b in range(NBUF):
                c = r * NBUF + b
                gather(c, b).wait()

                @pl.when(r > 0)
                def _():
                    write(c - NBUF, b).wait()

                transpose_chunk(b)
                write(c, b).start()

                @pl.when(c + NBUF < NCH)
                def _():
                    gather(c + NBUF, b).start()

            return carry

        lax.fori_loop(0, NCH // NBUF, round_body, 0)
        for b in range(NBUF):
            write(NCH - NBUF + b, b).wait()

    return embed


_embed = _make_embed()


def kernel(inputs, embedding_table):
    idx = (inputs.T.reshape(SEQ_LEN // 8, 8, SB, CHUNK)
           .transpose(0, 2, 1, 3).reshape(TOTAL).astype(jnp.int32))
    tab = lax.optimization_barrier(
        embedding_table.reshape(VOCAB_ROWS * EMB)
    ).reshape(VOCAB_ROWS, EMB)
    out_seb = _embed(tab, idx)
    return jnp.transpose(out_seb, (2, 0, 1))


# speedup vs baseline: 1.3423x; 1.1906x over previous
import functools

import jax
import jax.numpy as jnp
from jax import lax
from jax.experimental import pallas as pl
from jax.experimental.pallas import tpu as pltpu
from jax.experimental.pallas import tpu_sc as plsc

BATCH = 4096
SEQ_LEN = 200
EMB = 64
TOTAL = BATCH * SEQ_LEN
VOCAB_ROWS = 1000000

NC = 2
NS = 16
NW = NC * NS
ROWS_PER_W = TOTAL // NW
CHUNK = 128
NCH = ROWS_PER_W // CHUNK
NBUF = 4
SB = BATCH // CHUNK
LANES = 16


def _make_embed():
    mesh = plsc.VectorSubcoreMesh(core_axis_name="c", subcore_axis_name="s")

    @functools.partial(
        pl.kernel,
        mesh=mesh,
        out_type=jax.ShapeDtypeStruct((SEQ_LEN, EMB, BATCH), jnp.float32),
        scratch_types=[
            pltpu.VMEM((ROWS_PER_W,), jnp.int32),
            pltpu.VMEM((NBUF, CHUNK, EMB), jnp.float32),
            pltpu.VMEM((NBUF, EMB, CHUNK), jnp.float32),
            pltpu.SemaphoreType.DMA((NBUF,)),
            pltpu.SemaphoreType.DMA((NBUF,)),
        ],
        compiler_params=pltpu.CompilerParams(
            use_tc_tiling_on_sc=False, needs_layout_passes=False
        ),
    )
    def embed(table_hbm, idx_hbm, out_hbm, idx_v, wbufs, tbufs, gsem, wsem):
        wid = lax.axis_index("s") * NC + lax.axis_index("c")
        pltpu.sync_copy(idx_hbm.at[pl.ds(wid * ROWS_PER_W, ROWS_PER_W)], idx_v)
        q0 = wid * NCH

        def gather(c, b):
            return pltpu.make_async_copy(
                table_hbm.at[idx_v.at[pl.ds(c * CHUNK, CHUNK)]],
                wbufs.at[b],
                gsem.at[b],
            )

        def write(c, b):
            q = q0 + c
            s = 8 * (q // (SB * 8)) + (q % 8)
            b0 = CHUNK * ((q % (SB * 8)) // 8)
            return pltpu.make_async_copy(
                tbufs.at[b],
                out_hbm.at[s, :, pl.ds(b0, CHUNK)],
                wsem.at[b],
            )

        iota16 = lax.iota(jnp.int32, LANES)
        e_rows = [iota16 + q * LANES for q in range(EMB // LANES)]
        zeros16 = jnp.zeros((LANES,), jnp.int32)

        def transpose_chunk(b):
            @plsc.parallel_loop(0, CHUNK, unroll=8)
            def jrow(j):
                cols = zeros16 + j
                for q in range(EMB // LANES):
                    vals = wbufs[b, j, pl.ds(q * LANES, LANES)]
                    plsc.store_scatter(tbufs.at[b], [e_rows[q], cols], vals)

        for b in range(NBUF):
            gather(b, b).start()

        def round_body(r, carry):
            for b in range(NBUF):
                c = r * NBUF + b
                gather(c, b).wait()

                @pl.when(r > 0)
                def _():
                    write(c - NBUF, b).wait()

                transpose_chunk(b)
                write(c, b).start()

                @pl.when(c + NBUF < NCH)
                def _():
                    gather(c + NBUF, b).start()

            return carry

        lax.fori_loop(0, NCH // NBUF, round_body, 0)
        for b in range(NBUF):
            write(NCH - NBUF + b, b).wait()

    return embed


_embed = _make_embed()


def kernel(inputs, embedding_table):
    idx = (inputs.T.reshape(SEQ_LEN // 8, 8, SB, CHUNK)
           .transpose(0, 2, 1, 3).reshape(TOTAL).astype(jnp.int32))
    tab = lax.optimization_barrier(
        embedding_table.reshape(VOCAB_ROWS * EMB)
    ).reshape(VOCAB_ROWS, EMB)
    out_seb = _embed(tab, idx)
    return jnp.transpose(out_seb, (2, 0, 1))


# speedup vs baseline: 2.1254x; 1.5834x over previous
import functools

import jax
import jax.numpy as jnp
from jax import lax
from jax.experimental import pallas as pl
from jax.experimental.pallas import tpu as pltpu
from jax.experimental.pallas import tpu_sc as plsc

BATCH = 4096
SEQ_LEN = 200
EMB = 64
TOTAL = BATCH * SEQ_LEN
VOCAB_ROWS = 1000000

NC = 2
NS = 16
NW = NC * NS
ROWS_PER_W = TOTAL // NW
CHUNK = 128
NCH = ROWS_PER_W // CHUNK
NBUF = 4
SB = BATCH // CHUNK
LANES = 16


def _make_embed():
    mesh = plsc.VectorSubcoreMesh(core_axis_name="c", subcore_axis_name="s")

    @functools.partial(
        pl.kernel,
        mesh=mesh,
        out_type=jax.ShapeDtypeStruct((SEQ_LEN, EMB, BATCH), jnp.float32),
        scratch_types=[
            pltpu.VMEM((ROWS_PER_W,), jnp.int32),
            pltpu.VMEM((NBUF, CHUNK, EMB), jnp.float32),
            pltpu.VMEM((NBUF, EMB, CHUNK), jnp.float32),
            pltpu.SemaphoreType.DMA((NBUF,)),
            pltpu.SemaphoreType.DMA((NBUF,)),
        ],
        compiler_params=pltpu.CompilerParams(
            use_tc_tiling_on_sc=False, needs_layout_passes=False
        ),
    )
    def embed(table_hbm, idx_hbm, out_hbm, idx_v, wbufs, tbufs, gsem, wsem):
        wid = lax.axis_index("s") * NC + lax.axis_index("c")
        pltpu.sync_copy(idx_hbm.at[pl.ds(wid * ROWS_PER_W, ROWS_PER_W)], idx_v)
        q0 = wid * NCH

        def gather(c, b):
            return pltpu.make_async_copy(
                table_hbm.at[idx_v.at[pl.ds(c * CHUNK, CHUNK)]],
                wbufs.at[b],
                gsem.at[b],
            )

        def write(c, b):
            q = q0 + c
            s = 8 * (q // (SB * 8)) + (q % 8)
            b0 = CHUNK * ((q % (SB * 8)) // 8)
            return pltpu.make_async_copy(
                tbufs.at[b],
                out_hbm.at[s, :, pl.ds(b0, CHUNK)],
                wsem.at[b],
            )

        iota16 = lax.iota(jnp.int32, LANES)
        e_rows = [iota16 + q * LANES for q in range(EMB // LANES)]
        ones16 = jnp.full((LANES,), 1, jnp.int32)
        zeros16 = jnp.zeros((LANES,), jnp.int32)

        def transpose_chunk(b):
            @functools.partial(
                plsc.parallel_loop, 0, CHUNK, unroll=16, carry=zeros16
            )
            def jrow(j, cols):
                for q in range(EMB // LANES):
                    vals = wbufs[b, j, pl.ds(q * LANES, LANES)]
                    plsc.store_scatter(tbufs.at[b], [e_rows[q], cols], vals)
                return cols + ones16

        for b in range(NBUF):
            gather(b, b).start()

        def round_body(r, carry):
            for b in range(NBUF):
                c = r * NBUF + b
                gather(c, b).wait()

                @pl.when(r > 0)
                def _():
                    write(c - NBUF, b).wait()

                transpose_chunk(b)
                write(c, b).start()

                @pl.when(c + NBUF < NCH)
                def _():
                    gather(c + NBUF, b).start()

            return carry

        lax.fori_loop(0, NCH // NBUF, round_body, 0)
        for b in range(NBUF):
            write(NCH - NBUF + b, b).wait()

    return embed


_embed = _make_embed()


def kernel(inputs, embedding_table):
    idx = (inputs.T.reshape(SEQ_LEN // 8, 8, SB, CHUNK)
           .transpose(0, 2, 1, 3).reshape(TOTAL).astype(jnp.int32))
    tab = lax.optimization_barrier(
        embedding_table.reshape(VOCAB_ROWS * EMB)
    ).reshape(VOCAB_ROWS, EMB)
    out_seb = _embed(tab, idx)
    return jnp.transpose(out_seb, (2, 0, 1))


# speedup vs baseline: 2.6960x; 1.2684x over previous
import functools

import jax
import jax.numpy as jnp
from jax import lax
from jax.experimental import pallas as pl
from jax.experimental.pallas import tpu as pltpu
from jax.experimental.pallas import tpu_sc as plsc

BATCH = 4096
SEQ_LEN = 200
EMB = 64
TOTAL = BATCH * SEQ_LEN
VOCAB_ROWS = 1000000

NC = 2
NS = 16
NW = NC * NS
ROWS_PER_W = TOTAL // NW
CHUNK = 128
NCH = ROWS_PER_W // CHUNK
NBUF = 4
SB = BATCH // CHUNK
LANES = 16


def _make_embed():
    mesh = plsc.VectorSubcoreMesh(core_axis_name="c", subcore_axis_name="s")

    @functools.partial(
        pl.kernel,
        mesh=mesh,
        out_type=jax.ShapeDtypeStruct((SEQ_LEN, 8, SB, 8, CHUNK), jnp.float32),
        scratch_types=[
            pltpu.VMEM((ROWS_PER_W,), jnp.int32),
            pltpu.VMEM((NBUF, CHUNK, EMB), jnp.float32),
            pltpu.VMEM((NBUF, 8, 1, 8, CHUNK), jnp.float32),
            pltpu.SemaphoreType.DMA((NBUF,)),
            pltpu.SemaphoreType.DMA((NBUF,)),
        ],
        compiler_params=pltpu.CompilerParams(
            use_tc_tiling_on_sc=False, needs_layout_passes=False
        ),
    )
    def embed(table_hbm, idx_hbm, out_hbm, idx_v, wbufs, tbufs, gsem, wsem):
        wid = lax.axis_index("s") * NC + lax.axis_index("c")
        pltpu.sync_copy(idx_hbm.at[pl.ds(wid * ROWS_PER_W, ROWS_PER_W)], idx_v)
        q0 = wid * NCH

        def gather(c, b):
            return pltpu.make_async_copy(
                table_hbm.at[idx_v.at[pl.ds(c * CHUNK, CHUNK)]],
                wbufs.at[b],
                gsem.at[b],
            )

        def write(c, b):
            q = q0 + c
            s = 8 * (q // (SB * 8)) + (q % 8)
            tc = (q % (SB * 8)) // 8
            return pltpu.make_async_copy(
                tbufs.at[b],
                out_hbm.at[s, :, pl.ds(tc, 1)],
                wsem.at[b],
            )

        iota16 = lax.iota(jnp.int32, LANES)
        tr_rows = [iota16 // 8 + 2 * q for q in range(EMB // LANES)]
        sl_rows = iota16 % 8
        ones16 = jnp.full((LANES,), 1, jnp.int32)
        zeros16 = jnp.zeros((LANES,), jnp.int32)

        def transpose_chunk(b):
            @functools.partial(
                plsc.parallel_loop, 0, CHUNK, unroll=16, carry=zeros16
            )
            def jrow(j, cols):
                for q in range(EMB // LANES):
                    vals = wbufs[b, j, pl.ds(q * LANES, LANES)]
                    plsc.store_scatter(
                        tbufs.at[b],
                        [tr_rows[q], zeros16, sl_rows, cols],
                        vals,
                    )
                return cols + ones16

        for b in range(NBUF):
            gather(b, b).start()

        def round_body(r, carry):
            for b in range(NBUF):
                c = r * NBUF + b
                gather(c, b).wait()

                @pl.when(r > 0)
                def _():
                    write(c - NBUF, b).wait()

                transpose_chunk(b)
                write(c, b).start()

                @pl.when(c + NBUF < NCH)
                def _():
                    gather(c + NBUF, b).start()

            return carry

        lax.fori_loop(0, NCH // NBUF, round_body, 0)
        for b in range(NBUF):
            write(NCH - NBUF + b, b).wait()

    return embed


_embed = _make_embed()


def kernel(inputs, embedding_table):
    idx = (inputs.T.reshape(SEQ_LEN // 8, 8, SB, CHUNK)
           .transpose(0, 2, 1, 3).reshape(TOTAL).astype(jnp.int32))
    tab = lax.optimization_barrier(
        embedding_table.reshape(VOCAB_ROWS * EMB)
    ).reshape(VOCAB_ROWS, EMB)
    out5 = _embed(tab, idx)
    return out5.transpose(2, 4, 0, 1, 3).reshape(BATCH, SEQ_LEN, EMB)
